# Initial kernel scaffold; baseline (speedup 1.0000x reference)
import jax
import jax.numpy as jnp
from jax.experimental import pallas as pl


def kernel(agent_repr_1, ego_distance, ego_mask, ego_speed, Wq, bq, Wk, bk, Wv, bv, Weq, beq, Wek, bek, Wev, bev, Wd1, bd1, Wd2, bd2, ln_g, ln_b):
    raise NotImplementedError("write your pallas kernel here")


# speedup vs baseline: 16.8141x; 16.8141x over previous
import functools

import jax
import jax.numpy as jnp
from jax.experimental import pallas as pl
from jax.experimental.pallas import tpu as pltpu

_B, _N, _D = 64, 256, 256
_NH = 4
_HD = _D // _NH
_THR = 20.0
_KDEF = 4
_KMAX = 6
_NC = 7
_NCP = 8


def _select_body(dist_ref, speed_ref, idx_ref, cdist_ref, k_ref):
    d0 = dist_ref[...]
    close = jnp.sum((d0 < _THR).astype(jnp.float32), axis=1, keepdims=True)
    avg_density = jnp.mean(close) / d0.shape[1]
    avg_speed = jnp.mean(speed_ref[...])
    k = _KDEF + (avg_speed > 15.0).astype(jnp.int32)
    k = jnp.minimum(k, _KMAX)
    k = jnp.minimum(k + (avg_density > 0.5).astype(jnp.int32), _KMAX)
    k = jnp.minimum(k, d0.shape[1] - 1)
    k_ref[...] = jnp.full((1, 1), k, jnp.int32)

    iota_n = jax.lax.broadcasted_iota(jnp.int32, d0.shape, 1)
    work = d0
    for j in range(_NC):
        mval = jnp.min(work, axis=1, keepdims=True)
        cand = jnp.where(work == mval, iota_n, d0.shape[1])
        midx = jnp.min(cand, axis=1, keepdims=True)
        idx_ref[:, j:j + 1] = midx
        cdist_ref[:, j:j + 1] = mval
        work = jnp.where(iota_n == midx, jnp.inf, work)
    idx_ref[:, _NC:_NCP] = jnp.zeros((_B, _NCP - _NC), jnp.int32)
    cdist_ref[:, _NC:_NCP] = jnp.zeros((_B, _NCP - _NC), jnp.float32)


def _attn_body(idx_sref, k_sref, cd_sref,
               x_ref, distT_ref, maskT_ref,
               wq_ref, bq_ref, weq_ref, beq_ref,
               wk_ref, bk_ref, wv_ref, bv_ref,
               wd1t_ref, bd1_ref, wd2_ref, bd2_ref,
               lng_ref, lnb_ref, out_ref, cand_ref):
    b = pl.program_id(0)
    x = x_ref[0]
    cdims = (((1,), (1,)), ((), ()))

    qx = jax.lax.dot_general(x, wq_ref[...], cdims,
                             preferred_element_type=jnp.float32) + bq_ref[...]
    qe = jax.lax.dot_general(x, weq_ref[...], cdims,
                             preferred_element_type=jnp.float32) + beq_ref[...]
    lane = jax.lax.broadcasted_iota(jnp.int32, (_N, _B), 1)
    mcol = jnp.sum(jnp.where(lane == b, maskT_ref[...], 0.0),
                   axis=1, keepdims=True)
    q = qx + mcol * (qe - qx)

    for j in range(_NC):
        cand_ref[j:j + 1, :] = x_ref[0, pl.ds(idx_sref[b, j], 1), :]
    cand_ref[_NC:_NCP, :] = jnp.zeros((_NCP - _NC, _D), jnp.float32)
    cand = cand_ref[...]

    kc = jax.lax.dot_general(cand, wk_ref[...], cdims,
                             preferred_element_type=jnp.float32) + bk_ref[...]
    vc = jax.lax.dot_general(cand, wv_ref[...], cdims,
                             preferred_element_type=jnp.float32) + bv_ref[...]

    qd = jnp.sum(jnp.where(lane == b, distT_ref[...], 0.0),
                 axis=1, keepdims=True)
    w1a = wd1t_ref[0:1, :]
    w1b = wd1t_ref[1:2, :]
    bias_cols = []
    for j in range(_NCP):
        kd_j = cd_sref[b, j]
        h = jnp.maximum(qd * w1a + kd_j * w1b + bd1_ref[...], 0.0)
        bias_cols.append(
            jax.lax.dot_general(h, wd2_ref[...], cdims,
                                preferred_element_type=jnp.float32)
            + bd2_ref[...])

    rown = jax.lax.broadcasted_iota(jnp.int32, (_N, 1), 0)
    p = jnp.full((_N, 1), _N + 1, jnp.int32)
    for j in range(_NC):
        p = jnp.where(rown == idx_sref[b, j], j, p)
    jvec = jax.lax.broadcasted_iota(jnp.int32, (_N, _NCP), 1)
    k_scal = k_sref[0, 0]
    valid = (jvec != p) & ((jvec - (p < jvec).astype(jnp.int32)) < k_scal)

    inv_sqrt_hd = 1.0 / (_HD ** 0.5)
    outs = []
    for hgrp in range(_NH):
        sl = slice(hgrp * _HD, (hgrp + 1) * _HD)
        qh = q[:, sl]
        kh = kc[:, sl]
        vh = vc[:, sl]
        s = jax.lax.dot_general(qh, kh, cdims,
                                preferred_element_type=jnp.float32)
        s = s * inv_sqrt_hd
        bias_h = jnp.concatenate(
            [bias_cols[j][:, hgrp:hgrp + 1] for j in range(_NCP)], axis=1)
        s = s * bias_h
        s = jnp.where(valid, s, -1e30)
        m = jnp.max(s, axis=1, keepdims=True)
        e = jnp.exp(s - m)
        a = e / jnp.sum(e, axis=1, keepdims=True)
        outs.append(jax.lax.dot_general(a, vh, (((1,), (0,)), ((), ())),
                                        preferred_element_type=jnp.float32))
    attn = jnp.concatenate(outs, axis=1)

    xo = x + attn
    mu = jnp.mean(xo, axis=1, keepdims=True)
    var = jnp.mean((xo - mu) * (xo - mu), axis=1, keepdims=True)
    y = (xo - mu) * jax.lax.rsqrt(var + 1e-5)
    out_ref[0] = y * lng_ref[...] + lnb_ref[...]


@functools.partial(jax.jit, static_argnames=())
def kernel(agent_repr_1, ego_distance, ego_mask, ego_speed,
           Wq, bq, Wk, bk, Wv, bv, Weq, beq, Wek, bek, Wev, bev,
           Wd1, bd1, Wd2, bd2, ln_g, ln_b):
    b, n, d = agent_repr_1.shape

    top_idx, top_dist, k_arr = pl.pallas_call(
        _select_body,
        out_shape=(
            jax.ShapeDtypeStruct((b, _NCP), jnp.int32),
            jax.ShapeDtypeStruct((b, _NCP), jnp.float32),
            jax.ShapeDtypeStruct((1, 1), jnp.int32),
        ),
    )(ego_distance, ego_speed.reshape(1, b))

    distT = ego_distance.T
    maskT = ego_mask.astype(jnp.float32).T
    full = lambda shape: pl.BlockSpec(shape, lambda i, *_: (0,) * len(shape))

    grid_spec = pltpu.PrefetchScalarGridSpec(
        num_scalar_prefetch=3,
        grid=(b,),
        in_specs=[
            pl.BlockSpec((1, n, d), lambda i, *_: (i, 0, 0)),
            full((n, b)),
            full((n, b)),
            full((d, d)), full((1, d)),
            full((d, d)), full((1, d)),
            full((d, d)), full((1, d)),
            full((d, d)), full((1, d)),
            full((2, d // 4)), full((1, d // 4)),
            full((_NH, d // 4)), full((1, _NH)),
            full((1, d)), full((1, d)),
        ],
        out_specs=pl.BlockSpec((1, n, d), lambda i, *_: (i, 0, 0)),
        scratch_shapes=[pltpu.VMEM((_NCP, d), jnp.float32)],
    )

    out = pl.pallas_call(
        _attn_body,
        grid_spec=grid_spec,
        out_shape=jax.ShapeDtypeStruct((b, n, d), jnp.float32),
    )(top_idx, k_arr, top_dist,
      agent_repr_1, distT, maskT,
      Wq, bq.reshape(1, d), Weq, beq.reshape(1, d),
      Wk, bk.reshape(1, d), Wv, bv.reshape(1, d),
      Wd1.T, bd1.reshape(1, d // 4), Wd2, bd2.reshape(1, _NH),
      ln_g.reshape(1, d), ln_b.reshape(1, d))
    return out


# speedup vs baseline: 28.1365x; 1.6734x over previous
import functools

import jax
import jax.numpy as jnp
from jax.experimental import pallas as pl
from jax.experimental.pallas import tpu as pltpu

_B, _N, _D = 64, 256, 256
_NH = 4
_HD = _D // _NH
_THR = 20.0
_KDEF = 4
_KMAX = 6
_NC = 7
_NCP = 8
_HJ = _NH * _NCP


def _select_body(dist_ref, speed_ref, idx_ref, cdist_ref, k_ref):
    d0 = dist_ref[...]
    close = jnp.sum((d0 < _THR).astype(jnp.float32), axis=1, keepdims=True)
    avg_density = jnp.mean(close) / d0.shape[1]
    avg_speed = jnp.mean(speed_ref[...])
    k = _KDEF + (avg_speed > 15.0).astype(jnp.int32)
    k = jnp.minimum(k, _KMAX)
    k = jnp.minimum(k + (avg_density > 0.5).astype(jnp.int32), _KMAX)
    k = jnp.minimum(k, d0.shape[1] - 1)
    k_ref[...] = jnp.full((1, 1), k, jnp.int32)

    iota_n = jax.lax.broadcasted_iota(jnp.int32, d0.shape, 1)
    work = d0
    for j in range(_NC):
        mval = jnp.min(work, axis=1, keepdims=True)
        cand = jnp.where(work == mval, iota_n, d0.shape[1])
        midx = jnp.min(cand, axis=1, keepdims=True)
        idx_ref[:, j:j + 1] = midx
        cdist_ref[:, j:j + 1] = mval
        work = jnp.where(iota_n == midx, jnp.inf, work)
    idx_ref[:, _NC:_NCP] = jnp.zeros((_B, _NCP - _NC), jnp.int32)
    cdist_ref[:, _NC:_NCP] = jnp.zeros((_B, _NCP - _NC), jnp.float32)


def _attn_body(idx_sref, k_sref, cd_sref,
               x_ref, distT_ref, maskT_ref,
               wq_ref, bq_ref, weq_ref, beq_ref,
               wk_ref, bk_ref, wv_ref, bv_ref,
               w1a_ref, w1b_ref, bd1t_ref, wbig_ref, bd2big_ref,
               lng_ref, lnb_ref, out_ref, cand_ref):
    b = pl.program_id(0)
    x = x_ref[0]
    cdims = (((1,), (1,)), ((), ()))

    qx = jax.lax.dot_general(x, wq_ref[...], cdims,
                             preferred_element_type=jnp.float32) + bq_ref[...]
    qe = jax.lax.dot_general(x, weq_ref[...], cdims,
                             preferred_element_type=jnp.float32) + beq_ref[...]
    lane = jax.lax.broadcasted_iota(jnp.int32, (_N, _B), 1)
    mcol = jnp.sum(jnp.where(lane == b, maskT_ref[...], 0.0),
                   axis=1, keepdims=True)
    q = qx + mcol * (qe - qx)

    for j in range(_NC):
        cand_ref[j:j + 1, :] = x_ref[0, pl.ds(idx_sref[b, j], 1), :]
    cand_ref[_NC:_NCP, :] = jnp.zeros((_NCP - _NC, _D), jnp.float32)
    cand = cand_ref[...]

    kc = jax.lax.dot_general(cand, wk_ref[...], cdims,
                             preferred_element_type=jnp.float32) + bk_ref[...]
    vc = jax.lax.dot_general(cand, wv_ref[...], cdims,
                             preferred_element_type=jnp.float32) + bv_ref[...]

    hol = jax.lax.broadcasted_iota(jnp.int32, (_NCP, _D), 1) // _HD
    kcbig = jnp.concatenate(
        [jnp.where(hol == h, kc, 0.0) for h in range(_NH)], axis=0)
    vcbig = jnp.concatenate(
        [jnp.where(hol == h, vc, 0.0) for h in range(_NH)], axis=0)

    qd = jnp.sum(jnp.where(lane == b, distT_ref[...], 0.0),
                 axis=1, keepdims=True)
    kdvec = jnp.concatenate(
        [jnp.full((1, _HD), cd_sref[b, j], jnp.float32) for j in range(_NCP)],
        axis=1) * w1b_ref[...]
    h_all = jnp.maximum(qd * w1a_ref[...] + kdvec + bd1t_ref[...], 0.0)
    bias_all = jax.lax.dot_general(
        h_all, wbig_ref[...], (((1,), (0,)), ((), ())),
        preferred_element_type=jnp.float32) + bd2big_ref[...]

    inv_sqrt_hd = 1.0 / (_HD ** 0.5)
    s = jax.lax.dot_general(q, kcbig, cdims,
                            preferred_element_type=jnp.float32)
    s = s * inv_sqrt_hd * bias_all

    rown = jax.lax.broadcasted_iota(jnp.int32, (_N, 1), 0)
    p = jnp.full((_N, 1), _N + 1, jnp.int32)
    for j in range(_NC):
        p = jnp.where(rown == idx_sref[b, j], j, p)
    j32 = jax.lax.broadcasted_iota(jnp.int32, (_N, _HJ), 1) % _NCP
    k_scal = k_sref[0, 0]
    valid = (j32 != p) & ((j32 - (p < j32).astype(jnp.int32)) < k_scal)
    s = jnp.where(valid, s, -1e30)

    expand = (jax.lax.broadcasted_iota(jnp.int32, (_NH, _HJ), 1) // _NCP ==
              jax.lax.broadcasted_iota(jnp.int32, (_NH, _HJ), 0)
              ).astype(jnp.float32)
    m4 = jnp.concatenate(
        [jnp.max(s[:, h * _NCP:(h + 1) * _NCP], axis=1, keepdims=True)
         for h in range(_NH)], axis=1)
    m32 = jax.lax.dot_general(m4, expand, (((1,), (0,)), ((), ())),
                              preferred_element_type=jnp.float32)
    e = jnp.exp(s - m32)
    den4 = jax.lax.dot_general(e, expand, (((1,), (1,)), ((), ())),
                               preferred_element_type=jnp.float32)
    r32 = jax.lax.dot_general(1.0 / den4, expand, (((1,), (0,)), ((), ())),
                              preferred_element_type=jnp.float32)
    a = e * r32

    attn = jax.lax.dot_general(a, vcbig, (((1,), (0,)), ((), ())),
                               preferred_element_type=jnp.float32)

    xo = x + attn
    mu = jnp.mean(xo, axis=1, keepdims=True)
    var = jnp.mean((xo - mu) * (xo - mu), axis=1, keepdims=True)
    y = (xo - mu) * jax.lax.rsqrt(var + 1e-5)
    out_ref[0] = y * lng_ref[...] + lnb_ref[...]


@functools.partial(jax.jit, static_argnames=())
def kernel(agent_repr_1, ego_distance, ego_mask, ego_speed,
           Wq, bq, Wk, bk, Wv, bv, Weq, beq, Wek, bek, Wev, bev,
           Wd1, bd1, Wd2, bd2, ln_g, ln_b):
    b, n, d = agent_repr_1.shape
    dq = d // _NH

    top_idx, top_dist, k_arr = pl.pallas_call(
        _select_body,
        out_shape=(
            jax.ShapeDtypeStruct((b, _NCP), jnp.int32),
            jax.ShapeDtypeStruct((b, _NCP), jnp.float32),
            jax.ShapeDtypeStruct((1, 1), jnp.int32),
        ),
    )(ego_distance, ego_speed.reshape(1, b))

    distT = ego_distance.T
    maskT = ego_mask.astype(jnp.float32).T

    hd4 = Wd1.shape[0]
    w1a_t = jnp.tile(Wd1[:, 0], _NCP).reshape(1, _NCP * hd4)
    w1b_t = jnp.tile(Wd1[:, 1], _NCP).reshape(1, _NCP * hd4)
    bd1_t = jnp.tile(bd1, _NCP).reshape(1, _NCP * hd4)
    wbig = jnp.einsum('ch,jJ->jchJ', Wd2.T,
                      jnp.eye(_NCP, dtype=jnp.float32)
                      ).reshape(_NCP * hd4, _NH * _NCP)
    bd2big = jnp.repeat(bd2, _NCP).reshape(1, _NH * _NCP)

    full = lambda shape: pl.BlockSpec(shape, lambda i, *_: (0,) * len(shape))
    grid_spec = pltpu.PrefetchScalarGridSpec(
        num_scalar_prefetch=3,
        grid=(b,),
        in_specs=[
            pl.BlockSpec((1, n, d), lambda i, *_: (i, 0, 0)),
            full((n, b)),
            full((n, b)),
            full((d, d)), full((1, d)),
            full((d, d)), full((1, d)),
            full((d, d)), full((1, d)),
            full((d, d)), full((1, d)),
            full((1, _NCP * hd4)),
            full((1, _NCP * hd4)),
            full((1, _NCP * hd4)),
            full((_NCP * hd4, _HJ)),
            full((1, _HJ)),
            full((1, d)), full((1, d)),
        ],
        out_specs=pl.BlockSpec((1, n, d), lambda i, *_: (i, 0, 0)),
        scratch_shapes=[pltpu.VMEM((_NCP, d), jnp.float32)],
    )

    out = pl.pallas_call(
        _attn_body,
        grid_spec=grid_spec,
        out_shape=jax.ShapeDtypeStruct((b, n, d), jnp.float32),
    )(top_idx, k_arr, top_dist,
      agent_repr_1, distT, maskT,
      Wq, bq.reshape(1, d), Weq, beq.reshape(1, d),
      Wk, bk.reshape(1, d), Wv, bv.reshape(1, d),
      w1a_t, w1b_t, bd1_t, wbig, bd2big,
      ln_g.reshape(1, d), ln_b.reshape(1, d))
    return out
